# initial kernel scaffold (unmeasured)
import jax
import jax.numpy as jnp
from jax import lax
from jax.experimental import pallas as pl
from jax.experimental.pallas import tpu as pltpu


def kernel(partial, resid, gamma):
    _, M, D = partial.shape

    def comm_body(p_ref, recv_ref, send_sem, recv_sem):
        my_x = lax.axis_index("x")
        my_y = lax.axis_index("y")
        my_z = lax.axis_index("z")

        barrier_sem = pltpu.get_barrier_semaphore()
        pl.semaphore_signal(
            barrier_sem,
            inc=1,
            device_id=(my_x, 1 - my_y, my_z),
            device_id_type=pl.DeviceIdType.MESH,
        )
        pl.semaphore_wait(barrier_sem, 1)

        rdma = pltpu.make_async_remote_copy(
            src_ref=p_ref,
            dst_ref=recv_ref,
            send_sem=send_sem,
            recv_sem=recv_sem,
            device_id=(my_x, 1 - my_y, my_z),
            device_id_type=pl.DeviceIdType.MESH,
        )
        rdma.start()
        rdma.wait()

    recv = pl.pallas_call(
        comm_body,
        out_shape=jax.ShapeDtypeStruct((1, M, D), jnp.float32),
        in_specs=[pl.BlockSpec(memory_space=pltpu.ANY)],
        out_specs=pl.BlockSpec(memory_space=pltpu.ANY),
        scratch_shapes=[pltpu.SemaphoreType.DMA, pltpu.SemaphoreType.DMA],
        compiler_params=pltpu.CompilerParams(
            collective_id=0, has_side_effects=True
        ),
    )(partial)

    BLK = 256
    gamma2 = gamma.reshape(1, D)

    def ln_body(p_ref, q_ref, r_ref, g_ref, o_ref):
        y = p_ref[0] + q_ref[0] + r_ref[...]
        rms = jnp.sqrt(jnp.mean(y * y, axis=-1, keepdims=True) + 1e-6)
        o_ref[...] = (y / rms) * g_ref[...]

    return pl.pallas_call(
        ln_body,
        grid=(M // BLK,),
        in_specs=[
            pl.BlockSpec((1, BLK, D), lambda i: (0, i, 0)),
            pl.BlockSpec((1, BLK, D), lambda i: (0, i, 0)),
            pl.BlockSpec((BLK, D), lambda i: (i, 0)),
            pl.BlockSpec((1, D), lambda i: (0, 0)),
        ],
        out_specs=pl.BlockSpec((BLK, D), lambda i: (i, 0)),
        out_shape=jax.ShapeDtypeStruct((M, D), jnp.float32),
    )(partial, recv, resid, gamma2)


# baseline (device time: 807530 ns/iter reference)
import jax
import jax.numpy as jnp
from jax import lax
from jax.experimental import pallas as pl
from jax.experimental.pallas import tpu as pltpu


def kernel(partial, resid, gamma):
    _, M, D = partial.shape

    def comm_body(p_ref, recv_ref, send_sem, recv_sem):
        my_x = lax.axis_index("x")
        my_y = lax.axis_index("y")
        my_z = lax.axis_index("z")

        barrier_sem = pltpu.get_barrier_semaphore()
        pl.semaphore_signal(
            barrier_sem,
            inc=1,
            device_id=(my_x, 1 - my_y, my_z),
            device_id_type=pl.DeviceIdType.MESH,
        )
        pl.semaphore_wait(barrier_sem, 1)

        rdma = pltpu.make_async_remote_copy(
            src_ref=p_ref,
            dst_ref=recv_ref,
            send_sem=send_sem,
            recv_sem=recv_sem,
            device_id=(my_x, 1 - my_y, my_z),
            device_id_type=pl.DeviceIdType.MESH,
        )
        rdma.start()
        rdma.wait()

    recv = pl.pallas_call(
        comm_body,
        out_shape=jax.ShapeDtypeStruct((1, M, D), jnp.float32),
        in_specs=[pl.BlockSpec(memory_space=pl.ANY)],
        out_specs=pl.BlockSpec(memory_space=pl.ANY),
        scratch_shapes=[pltpu.SemaphoreType.DMA, pltpu.SemaphoreType.DMA],
        compiler_params=pltpu.CompilerParams(
            collective_id=0, has_side_effects=True
        ),
    )(partial)

    BLK = 128
    gamma2 = gamma.reshape(1, D)

    def ln_body(p_ref, q_ref, r_ref, g_ref, o_ref):
        y = p_ref[0] + q_ref[0] + r_ref[...]
        rms = jnp.sqrt(jnp.mean(y * y, axis=-1, keepdims=True) + 1e-6)
        o_ref[...] = (y / rms) * g_ref[...]

    return pl.pallas_call(
        ln_body,
        grid=(M // BLK,),
        in_specs=[
            pl.BlockSpec((1, BLK, D), lambda i: (0, i, 0)),
            pl.BlockSpec((1, BLK, D), lambda i: (0, i, 0)),
            pl.BlockSpec((BLK, D), lambda i: (i, 0)),
            pl.BlockSpec((1, D), lambda i: (0, 0)),
        ],
        out_specs=pl.BlockSpec((BLK, D), lambda i: (i, 0)),
        out_shape=jax.ShapeDtypeStruct((M, D), jnp.float32),
    )(partial, recv, resid, gamma2)


# device time: 438000 ns/iter; 1.8437x vs baseline; 1.8437x over previous
import jax
import jax.numpy as jnp
from jax import lax
from jax.experimental import pallas as pl
from jax.experimental.pallas import tpu as pltpu

CH = 128
N_CH = 16


def kernel(partial, resid, gamma):
    _, M, D = partial.shape
    HALF = M // 2
    assert N_CH * CH == HALF

    gamma2 = gamma.reshape(1, D)

    def body(
        p_ref,
        res_ref,
        g_ref,
        o_ref,
        recv_hbm,
        own_v,
        prt_v,
        res_v,
        out_v,
        y_send, y_recv, z_send, z_recv,
        cp_own, cp_prt, cp_res,
        st_sem,
    ):
        my_x = lax.axis_index("x")
        my_y = lax.axis_index("y")
        my_z = lax.axis_index("z")
        y_nbr = (my_x, 1 - my_y, my_z)
        z_nbr = (my_x, my_y, 1 - my_z)
        base = my_z * HALF

        barrier_sem = pltpu.get_barrier_semaphore()
        for nbr in (y_nbr, z_nbr):
            pl.semaphore_signal(
                barrier_sem, inc=1, device_id=nbr,
                device_id_type=pl.DeviceIdType.MESH,
            )
        pl.semaphore_wait(barrier_sem, 2)

        y_rdmas = []
        for c in range(N_CH):
            rows = pl.ds(base + c * CH, CH)
            r = pltpu.make_async_remote_copy(
                src_ref=p_ref.at[0, rows, :],
                dst_ref=recv_hbm.at[pl.ds(c * CH, CH), :],
                send_sem=y_send.at[c],
                recv_sem=y_recv.at[c],
                device_id=y_nbr,
                device_id_type=pl.DeviceIdType.MESH,
            )
            r.start()
            y_rdmas.append(r)

        z_rdmas = [None] * N_CH
        st_copies = [None] * N_CH
        for c in range(N_CH):
            slot = c % 2
            rows = pl.ds(base + c * CH, CH)

            cp1 = pltpu.make_async_copy(p_ref.at[0, rows, :], own_v, cp_own)
            cp2 = pltpu.make_async_copy(res_ref.at[rows, :], res_v, cp_res)
            cp1.start()
            cp2.start()

            y_rdmas[c].wait_recv()
            y_rdmas[c].wait_send()
            cp3 = pltpu.make_async_copy(
                recv_hbm.at[pl.ds(c * CH, CH), :], prt_v, cp_prt
            )
            cp3.start()
            cp1.wait()
            cp2.wait()
            cp3.wait()

            if c >= 2:
                z_rdmas[c - 2].wait_send()
                st_copies[c - 2].wait()

            y = own_v[...] + prt_v[...] + res_v[...]
            rms = jnp.sqrt(jnp.mean(y * y, axis=-1, keepdims=True) + 1e-6)
            out_v[slot] = (y / rms) * g_ref[...]

            st = pltpu.make_async_copy(
                out_v.at[slot], o_ref.at[rows, :], st_sem.at[slot]
            )
            st.start()
            st_copies[c] = st
            zr = pltpu.make_async_remote_copy(
                src_ref=out_v.at[slot],
                dst_ref=o_ref.at[rows, :],
                send_sem=z_send.at[c],
                recv_sem=z_recv.at[c],
                device_id=z_nbr,
                device_id_type=pl.DeviceIdType.MESH,
            )
            zr.start()
            z_rdmas[c] = zr

        for c in (N_CH - 2, N_CH - 1):
            z_rdmas[c].wait_send()
            st_copies[c].wait()
        for c in range(N_CH):
            z_rdmas[c].wait_recv()

    out, _ = pl.pallas_call(
        body,
        out_shape=[
            jax.ShapeDtypeStruct((M, D), jnp.float32),
            jax.ShapeDtypeStruct((HALF, D), jnp.float32),
        ],
        in_specs=[
            pl.BlockSpec(memory_space=pl.ANY),
            pl.BlockSpec(memory_space=pl.ANY),
            pl.BlockSpec(memory_space=pltpu.VMEM),
        ],
        out_specs=[
            pl.BlockSpec(memory_space=pl.ANY),
            pl.BlockSpec(memory_space=pl.ANY),
        ],
        scratch_shapes=[
            pltpu.VMEM((CH, D), jnp.float32),
            pltpu.VMEM((CH, D), jnp.float32),
            pltpu.VMEM((CH, D), jnp.float32),
            pltpu.VMEM((2, CH, D), jnp.float32),
            pltpu.SemaphoreType.DMA((N_CH,)),
            pltpu.SemaphoreType.DMA((N_CH,)),
            pltpu.SemaphoreType.DMA((N_CH,)),
            pltpu.SemaphoreType.DMA((N_CH,)),
            pltpu.SemaphoreType.DMA,
            pltpu.SemaphoreType.DMA,
            pltpu.SemaphoreType.DMA,
            pltpu.SemaphoreType.DMA((2,)),
        ],
        compiler_params=pltpu.CompilerParams(
            collective_id=0, has_side_effects=True
        ),
    )(partial, resid, gamma2)
    return out


# device time: 436059 ns/iter; 1.8519x vs baseline; 1.0045x over previous
import jax
import jax.numpy as jnp
from jax import lax
from jax.experimental import pallas as pl
from jax.experimental.pallas import tpu as pltpu

CH = 128
N_CH = 16


def kernel(partial, resid, gamma):
    _, M, D = partial.shape
    HALF = M // 2
    assert N_CH * CH == HALF

    gamma2 = gamma.reshape(1, D)

    def body(
        p_ref,
        res_ref,
        g_ref,
        o_ref,
        recv_v,
        own_v,
        res_v,
        out_v,
        y_send, y_recv, z_send, z_recv,
        cp_own, cp_res,
        st_sem,
    ):
        my_x = lax.axis_index("x")
        my_y = lax.axis_index("y")
        my_z = lax.axis_index("z")
        y_nbr = (my_x, 1 - my_y, my_z)
        z_nbr = (my_x, my_y, 1 - my_z)
        base = my_z * HALF

        barrier_sem = pltpu.get_barrier_semaphore()
        for nbr in (y_nbr, z_nbr):
            pl.semaphore_signal(
                barrier_sem, inc=1, device_id=nbr,
                device_id_type=pl.DeviceIdType.MESH,
            )
        pl.semaphore_wait(barrier_sem, 2)

        y_rdmas = []
        for c in range(N_CH):
            rows = pl.ds(base + c * CH, CH)
            r = pltpu.make_async_remote_copy(
                src_ref=p_ref.at[0, rows, :],
                dst_ref=recv_v.at[c],
                send_sem=y_send.at[c],
                recv_sem=y_recv.at[c],
                device_id=y_nbr,
                device_id_type=pl.DeviceIdType.MESH,
            )
            r.start()
            y_rdmas.append(r)

        def load(c):
            slot = c % 2
            rows = pl.ds(base + c * CH, CH)
            cp1 = pltpu.make_async_copy(
                p_ref.at[0, rows, :], own_v.at[slot], cp_own.at[slot]
            )
            cp2 = pltpu.make_async_copy(
                res_ref.at[rows, :], res_v.at[slot], cp_res.at[slot]
            )
            cp1.start()
            cp2.start()
            return cp1, cp2

        pending = load(0)
        z_rdmas = [None] * N_CH
        st_copies = [None] * N_CH
        for c in range(N_CH):
            slot = c % 2
            rows = pl.ds(base + c * CH, CH)

            pending[0].wait()
            pending[1].wait()
            if c + 1 < N_CH:
                pending = load(c + 1)

            y_rdmas[c].wait_recv()
            y_rdmas[c].wait_send()

            if c >= 2:
                z_rdmas[c - 2].wait_send()
                st_copies[c - 2].wait()

            y = own_v[slot] + recv_v[c] + res_v[slot]
            rms = jnp.sqrt(jnp.mean(y * y, axis=-1, keepdims=True) + 1e-6)
            out_v[slot] = (y / rms) * g_ref[...]

            st = pltpu.make_async_copy(
                out_v.at[slot], o_ref.at[rows, :], st_sem.at[slot]
            )
            st.start()
            st_copies[c] = st
            zr = pltpu.make_async_remote_copy(
                src_ref=out_v.at[slot],
                dst_ref=o_ref.at[rows, :],
                send_sem=z_send.at[c],
                recv_sem=z_recv.at[c],
                device_id=z_nbr,
                device_id_type=pl.DeviceIdType.MESH,
            )
            zr.start()
            z_rdmas[c] = zr

        for c in (N_CH - 2, N_CH - 1):
            z_rdmas[c].wait_send()
            st_copies[c].wait()
        for c in range(N_CH):
            z_rdmas[c].wait_recv()

    return pl.pallas_call(
        body,
        out_shape=jax.ShapeDtypeStruct((M, D), jnp.float32),
        in_specs=[
            pl.BlockSpec(memory_space=pl.ANY),
            pl.BlockSpec(memory_space=pl.ANY),
            pl.BlockSpec(memory_space=pltpu.VMEM),
        ],
        out_specs=pl.BlockSpec(memory_space=pl.ANY),
        scratch_shapes=[
            pltpu.VMEM((N_CH, CH, D), jnp.float32),
            pltpu.VMEM((2, CH, D), jnp.float32),
            pltpu.VMEM((2, CH, D), jnp.float32),
            pltpu.VMEM((2, CH, D), jnp.float32),
            pltpu.SemaphoreType.DMA((N_CH,)),
            pltpu.SemaphoreType.DMA((N_CH,)),
            pltpu.SemaphoreType.DMA((N_CH,)),
            pltpu.SemaphoreType.DMA((N_CH,)),
            pltpu.SemaphoreType.DMA((2,)),
            pltpu.SemaphoreType.DMA((2,)),
            pltpu.SemaphoreType.DMA((2,)),
        ],
        compiler_params=pltpu.CompilerParams(
            collective_id=0,
            has_side_effects=True,
            vmem_limit_bytes=100 * 1024 * 1024,
        ),
    )(partial, resid, gamma2)


# device time: 348721 ns/iter; 2.3157x vs baseline; 1.2505x over previous
import jax
import jax.numpy as jnp
from jax import lax
from jax.experimental import pallas as pl
from jax.experimental.pallas import tpu as pltpu

CH = 128
N_CH = 8


def kernel(partial, resid, gamma):
    _, M, D = partial.shape
    QTR = M // 4
    assert N_CH * CH == QTR

    gamma2 = gamma.reshape(1, D)

    def body(
        p_ref,
        res_ref,
        g_ref,
        o_ref,
        recv_v,
        own_v,
        res_v,
        out_v,
        y_send, y_recv,
        sx_send, sz_send,
        rx_recv, rz_recv,
        f_send, d_recv,
        cp_own, cp_res,
        st_sem,
    ):
        my_x = lax.axis_index("x")
        my_y = lax.axis_index("y")
        my_z = lax.axis_index("z")
        y_nbr = (my_x, 1 - my_y, my_z)
        x_nbr = (1 - my_x, my_y, my_z)
        z_nbr = (my_x, my_y, 1 - my_z)

        def qbase(x, z):
            return (2 * x + z) * QTR

        base = qbase(my_x, my_z)
        base_xq = qbase(1 - my_x, my_z)
        base_zq = qbase(my_x, 1 - my_z)
        base_dq = qbase(1 - my_x, 1 - my_z)

        barrier_sem = pltpu.get_barrier_semaphore()
        for nbr in (y_nbr, x_nbr, z_nbr):
            pl.semaphore_signal(
                barrier_sem, inc=1, device_id=nbr,
                device_id_type=pl.DeviceIdType.MESH,
            )
        pl.semaphore_wait(barrier_sem, 3)

        y_rdmas = []
        for c in range(N_CH):
            rows = pl.ds(base + c * CH, CH)
            r = pltpu.make_async_remote_copy(
                src_ref=p_ref.at[0, rows, :],
                dst_ref=recv_v.at[c],
                send_sem=y_send.at[c],
                recv_sem=y_recv.at[c],
                device_id=y_nbr,
                device_id_type=pl.DeviceIdType.MESH,
            )
            r.start()
            y_rdmas.append(r)

        def load(c):
            slot = c % 2
            rows = pl.ds(base + c * CH, CH)
            cp1 = pltpu.make_async_copy(
                p_ref.at[0, rows, :], own_v.at[slot], cp_own.at[slot]
            )
            cp2 = pltpu.make_async_copy(
                res_ref.at[rows, :], res_v.at[slot], cp_res.at[slot]
            )
            cp1.start()
            cp2.start()
            return cp1, cp2

        def inbound(rbase, sem, c):
            rows = pl.ds(rbase + c * CH, CH)
            return pltpu.make_async_remote_copy(
                src_ref=o_ref.at[rows, :],
                dst_ref=o_ref.at[rows, :],
                send_sem=f_send.at[c],
                recv_sem=sem.at[c],
                device_id=y_nbr,
                device_id_type=pl.DeviceIdType.MESH,
            )

        rx_waited = [False] * N_CH
        rz_waited = [False] * N_CH
        fwd_rdmas = [None] * N_CH

        def forward(k):
            if k % 2 == 0:
                inbound(base_zq, rz_recv, k).wait_recv()
                rz_waited[k] = True
                rows = pl.ds(base_zq + k * CH, CH)
                dst = x_nbr
            else:
                inbound(base_xq, rx_recv, k).wait_recv()
                rx_waited[k] = True
                rows = pl.ds(base_xq + k * CH, CH)
                dst = z_nbr
            fr = pltpu.make_async_remote_copy(
                src_ref=o_ref.at[rows, :],
                dst_ref=o_ref.at[rows, :],
                send_sem=f_send.at[k],
                recv_sem=d_recv.at[k],
                device_id=dst,
                device_id_type=pl.DeviceIdType.MESH,
            )
            fr.start()
            fwd_rdmas[k] = fr

        pending = load(0)
        sx_rdmas = [None] * N_CH
        sz_rdmas = [None] * N_CH
        st_copies = [None] * N_CH
        for c in range(N_CH):
            slot = c % 2
            rows = pl.ds(base + c * CH, CH)

            pending[0].wait()
            pending[1].wait()
            if c + 1 < N_CH:
                pending = load(c + 1)

            y_rdmas[c].wait_recv()
            y_rdmas[c].wait_send()

            if c >= 2:
                sx_rdmas[c - 2].wait_send()
                sz_rdmas[c - 2].wait_send()
                st_copies[c - 2].wait()

            y = own_v[slot] + recv_v[c] + res_v[slot]
            rms = jnp.sqrt(jnp.mean(y * y, axis=-1, keepdims=True) + 1e-6)
            out_v[slot] = (y / rms) * g_ref[...]

            st = pltpu.make_async_copy(
                out_v.at[slot], o_ref.at[rows, :], st_sem.at[slot]
            )
            st.start()
            st_copies[c] = st
            sx = pltpu.make_async_remote_copy(
                src_ref=out_v.at[slot],
                dst_ref=o_ref.at[rows, :],
                send_sem=sx_send.at[c],
                recv_sem=rx_recv.at[c],
                device_id=x_nbr,
                device_id_type=pl.DeviceIdType.MESH,
            )
            sx.start()
            sx_rdmas[c] = sx
            sz = pltpu.make_async_remote_copy(
                src_ref=out_v.at[slot],
                dst_ref=o_ref.at[rows, :],
                send_sem=sz_send.at[c],
                recv_sem=rz_recv.at[c],
                device_id=z_nbr,
                device_id_type=pl.DeviceIdType.MESH,
            )
            sz.start()
            sz_rdmas[c] = sz

            if c >= 1:
                forward(c - 1)
        forward(N_CH - 1)

        for c in (N_CH - 2, N_CH - 1):
            sx_rdmas[c].wait_send()
            sz_rdmas[c].wait_send()
            st_copies[c].wait()
        for c in range(N_CH):
            if not rx_waited[c]:
                inbound(base_xq, rx_recv, c).wait_recv()
            if not rz_waited[c]:
                inbound(base_zq, rz_recv, c).wait_recv()
            inbound(base_dq, d_recv, c).wait_recv()
        for c in range(N_CH):
            fwd_rdmas[c].wait_send()

    return pl.pallas_call(
        body,
        out_shape=jax.ShapeDtypeStruct((M, D), jnp.float32),
        in_specs=[
            pl.BlockSpec(memory_space=pl.ANY),
            pl.BlockSpec(memory_space=pl.ANY),
            pl.BlockSpec(memory_space=pltpu.VMEM),
        ],
        out_specs=pl.BlockSpec(memory_space=pl.ANY),
        scratch_shapes=[
            pltpu.VMEM((N_CH, CH, D), jnp.float32),
            pltpu.VMEM((2, CH, D), jnp.float32),
            pltpu.VMEM((2, CH, D), jnp.float32),
            pltpu.VMEM((2, CH, D), jnp.float32),
            pltpu.SemaphoreType.DMA((N_CH,)),
            pltpu.SemaphoreType.DMA((N_CH,)),
            pltpu.SemaphoreType.DMA((N_CH,)),
            pltpu.SemaphoreType.DMA((N_CH,)),
            pltpu.SemaphoreType.DMA((N_CH,)),
            pltpu.SemaphoreType.DMA((N_CH,)),
            pltpu.SemaphoreType.DMA((N_CH,)),
            pltpu.SemaphoreType.DMA((N_CH,)),
            pltpu.SemaphoreType.DMA((2,)),
            pltpu.SemaphoreType.DMA((2,)),
            pltpu.SemaphoreType.DMA((2,)),
        ],
        compiler_params=pltpu.CompilerParams(
            collective_id=0,
            has_side_effects=True,
            vmem_limit_bytes=100 * 1024 * 1024,
        ),
    )(partial, resid, gamma2)


# device time: 337339 ns/iter; 2.3938x vs baseline; 1.0337x over previous
import jax
import jax.numpy as jnp
from jax import lax
from jax.experimental import pallas as pl
from jax.experimental.pallas import tpu as pltpu

CH = 64
N_CH = 16


def kernel(partial, resid, gamma):
    _, M, D = partial.shape
    QTR = M // 4
    assert N_CH * CH == QTR

    gamma2 = gamma.reshape(1, D)

    def body(
        p_ref,
        res_ref,
        g_ref,
        o_ref,
        recv_v,
        own_v,
        res_v,
        out_v,
        y_send, y_recv,
        sx_send, sz_send,
        rx_recv, rz_recv,
        f_send, d_recv,
        cp_own, cp_res,
        st_sem,
    ):
        my_x = lax.axis_index("x")
        my_y = lax.axis_index("y")
        my_z = lax.axis_index("z")
        y_nbr = (my_x, 1 - my_y, my_z)
        x_nbr = (1 - my_x, my_y, my_z)
        z_nbr = (my_x, my_y, 1 - my_z)

        def qbase(x, z):
            return (2 * x + z) * QTR

        base = qbase(my_x, my_z)
        base_xq = qbase(1 - my_x, my_z)
        base_zq = qbase(my_x, 1 - my_z)
        base_dq = qbase(1 - my_x, 1 - my_z)

        barrier_sem = pltpu.get_barrier_semaphore()
        for nbr in (y_nbr, x_nbr, z_nbr):
            pl.semaphore_signal(
                barrier_sem, inc=1, device_id=nbr,
                device_id_type=pl.DeviceIdType.MESH,
            )
        pl.semaphore_wait(barrier_sem, 3)

        y_rdmas = []
        for c in range(N_CH):
            rows = pl.ds(base + c * CH, CH)
            r = pltpu.make_async_remote_copy(
                src_ref=p_ref.at[0, rows, :],
                dst_ref=recv_v.at[c],
                send_sem=y_send.at[c],
                recv_sem=y_recv.at[c],
                device_id=y_nbr,
                device_id_type=pl.DeviceIdType.MESH,
            )
            r.start()
            y_rdmas.append(r)

        def load(c):
            slot = c % 2
            rows = pl.ds(base + c * CH, CH)
            cp1 = pltpu.make_async_copy(
                p_ref.at[0, rows, :], own_v.at[slot], cp_own.at[slot]
            )
            cp2 = pltpu.make_async_copy(
                res_ref.at[rows, :], res_v.at[slot], cp_res.at[slot]
            )
            cp1.start()
            cp2.start()
            return cp1, cp2

        def inbound(rbase, sem, c):
            rows = pl.ds(rbase + c * CH, CH)
            return pltpu.make_async_remote_copy(
                src_ref=o_ref.at[rows, :],
                dst_ref=o_ref.at[rows, :],
                send_sem=f_send.at[c],
                recv_sem=sem.at[c],
                device_id=y_nbr,
                device_id_type=pl.DeviceIdType.MESH,
            )

        rx_waited = [False] * N_CH
        rz_waited = [False] * N_CH
        fwd_rdmas = [None] * N_CH

        def forward(k):
            if k % 2 == 0:
                inbound(base_zq, rz_recv, k).wait_recv()
                rz_waited[k] = True
                rows = pl.ds(base_zq + k * CH, CH)
                dst = x_nbr
            else:
                inbound(base_xq, rx_recv, k).wait_recv()
                rx_waited[k] = True
                rows = pl.ds(base_xq + k * CH, CH)
                dst = z_nbr
            fr = pltpu.make_async_remote_copy(
                src_ref=o_ref.at[rows, :],
                dst_ref=o_ref.at[rows, :],
                send_sem=f_send.at[k],
                recv_sem=d_recv.at[k],
                device_id=dst,
                device_id_type=pl.DeviceIdType.MESH,
            )
            fr.start()
            fwd_rdmas[k] = fr

        pending = load(0)
        sx_rdmas = [None] * N_CH
        sz_rdmas = [None] * N_CH
        st_copies = [None] * N_CH
        for c in range(N_CH):
            slot = c % 2
            rows = pl.ds(base + c * CH, CH)

            pending[0].wait()
            pending[1].wait()
            if c + 1 < N_CH:
                pending = load(c + 1)

            y_rdmas[c].wait_recv()
            y_rdmas[c].wait_send()

            if c >= 2:
                sx_rdmas[c - 2].wait_send()
                sz_rdmas[c - 2].wait_send()
                st_copies[c - 2].wait()

            y = own_v[slot] + recv_v[c] + res_v[slot]
            rms = jnp.sqrt(jnp.mean(y * y, axis=-1, keepdims=True) + 1e-6)
            out_v[slot] = (y / rms) * g_ref[...]

            st = pltpu.make_async_copy(
                out_v.at[slot], o_ref.at[rows, :], st_sem.at[slot]
            )
            st.start()
            st_copies[c] = st
            sx = pltpu.make_async_remote_copy(
                src_ref=out_v.at[slot],
                dst_ref=o_ref.at[rows, :],
                send_sem=sx_send.at[c],
                recv_sem=rx_recv.at[c],
                device_id=x_nbr,
                device_id_type=pl.DeviceIdType.MESH,
            )
            sx.start()
            sx_rdmas[c] = sx
            sz = pltpu.make_async_remote_copy(
                src_ref=out_v.at[slot],
                dst_ref=o_ref.at[rows, :],
                send_sem=sz_send.at[c],
                recv_sem=rz_recv.at[c],
                device_id=z_nbr,
                device_id_type=pl.DeviceIdType.MESH,
            )
            sz.start()
            sz_rdmas[c] = sz

            if c >= 1:
                forward(c - 1)
        forward(N_CH - 1)

        for c in (N_CH - 2, N_CH - 1):
            sx_rdmas[c].wait_send()
            sz_rdmas[c].wait_send()
            st_copies[c].wait()
        for c in range(N_CH):
            if not rx_waited[c]:
                inbound(base_xq, rx_recv, c).wait_recv()
            if not rz_waited[c]:
                inbound(base_zq, rz_recv, c).wait_recv()
            inbound(base_dq, d_recv, c).wait_recv()
        for c in range(N_CH):
            fwd_rdmas[c].wait_send()

    return pl.pallas_call(
        body,
        out_shape=jax.ShapeDtypeStruct((M, D), jnp.float32),
        in_specs=[
            pl.BlockSpec(memory_space=pl.ANY),
            pl.BlockSpec(memory_space=pl.ANY),
            pl.BlockSpec(memory_space=pltpu.VMEM),
        ],
        out_specs=pl.BlockSpec(memory_space=pl.ANY),
        scratch_shapes=[
            pltpu.VMEM((N_CH, CH, D), jnp.float32),
            pltpu.VMEM((2, CH, D), jnp.float32),
            pltpu.VMEM((2, CH, D), jnp.float32),
            pltpu.VMEM((2, CH, D), jnp.float32),
            pltpu.SemaphoreType.DMA((N_CH,)),
            pltpu.SemaphoreType.DMA((N_CH,)),
            pltpu.SemaphoreType.DMA((N_CH,)),
            pltpu.SemaphoreType.DMA((N_CH,)),
            pltpu.SemaphoreType.DMA((N_CH,)),
            pltpu.SemaphoreType.DMA((N_CH,)),
            pltpu.SemaphoreType.DMA((N_CH,)),
            pltpu.SemaphoreType.DMA((N_CH,)),
            pltpu.SemaphoreType.DMA((2,)),
            pltpu.SemaphoreType.DMA((2,)),
            pltpu.SemaphoreType.DMA((2,)),
        ],
        compiler_params=pltpu.CompilerParams(
            collective_id=0,
            has_side_effects=True,
            vmem_limit_bytes=100 * 1024 * 1024,
        ),
    )(partial, resid, gamma2)
